# baseline (speedup 1.0000x reference)
import jax
import jax.numpy as jnp
from jax import lax
from jax.experimental import pallas as pl
from jax.experimental.pallas import tpu as pltpu
from jax.experimental.pallas import tpu_sc as plsc

N = 10000
E = 320000
D = 128
NOUT = 32
SLOPE = 0.2

NC = 2
NS = 16
LL = 16
NW = NC * NS
EPW = E // NW
CHUNK = 400
NCHUNK = EPW // CHUNK
NGRP = CHUNK // LL


def _uv_body(x_ref, w_ref, b_ref, u_ref, v_ref):
    uv = lax.dot_general(x_ref[...], w_ref[...], (((1,), (0,)), ((), ())),
                         preferred_element_type=jnp.float32)
    u_ref[...] = uv[:, :NOUT] + b_ref[...]
    v_ref[...] = uv[:, NOUT:]


def _make_uv(x, w_cat, b2d):
    blk = 1000
    return pl.pallas_call(
        _uv_body,
        grid=(N // blk,),
        in_specs=[
            pl.BlockSpec((blk, D), lambda i: (i, 0)),
            pl.BlockSpec((D, 2 * NOUT), lambda i: (0, 0)),
            pl.BlockSpec((1, NOUT), lambda i: (0, 0)),
        ],
        out_specs=[
            pl.BlockSpec((blk, NOUT), lambda i: (i, 0)),
            pl.BlockSpec((blk, NOUT), lambda i: (i, 0)),
        ],
        out_shape=[
            jax.ShapeDtypeStruct((N, NOUT), jnp.float32),
            jax.ShapeDtypeStruct((N, NOUT), jnp.float32),
        ],
    )(x, w_cat, b2d)


_MESH = plsc.VectorSubcoreMesh(core_axis_name="c", subcore_axis_name="s",
                               num_cores=NC, num_subcores=NS)


def _edge_body(u_hbm, v_hbm, src_hbm, dst_hbm, arep_hbm,
               ex_hbm, parts_hbm,
               srcall, dstall, gu, gv, exw, arv, zb, shared, su, sv,
               semu, semv, semsc):
    c = lax.axis_index("c")
    s = lax.axis_index("s")
    wid = s * NC + c

    rpt = N // NS
    cu0 = pltpu.async_copy(u_hbm.at[pl.ds(s * rpt, rpt)],
                           su.at[pl.ds(s * rpt, rpt)], semu.at[0])
    cv0 = pltpu.async_copy(v_hbm.at[pl.ds(s * rpt, rpt)],
                           sv.at[pl.ds(s * rpt, rpt)], semv.at[0])

    @pl.when(s == 0)
    def _():
        @pl.loop(0, CHUNK // LL)
        def _(i):
            zb[pl.ds(i * LL, LL)] = jnp.zeros((LL,), jnp.float32)

        @pl.loop(0, N // CHUNK)
        def _(i):
            pltpu.sync_copy(zb, shared.at[pl.ds(i * CHUNK, CHUNK)])

    pltpu.sync_copy(arep_hbm, arv)
    pltpu.sync_copy(src_hbm.at[wid], srcall)
    pltpu.sync_copy(dst_hbm.at[wid], dstall)
    cu0.wait()
    cv0.wait()
    plsc.subcore_barrier()

    iot = lax.iota(jnp.int32, LL)
    avals = [arv[k, :] for k in range(NOUT)]

    def issue(j, slot):
        pltpu.async_copy(su.at[srcall.at[j]], gu.at[slot], semu.at[slot])
        pltpu.async_copy(sv.at[dstall.at[j]], gv.at[slot], semv.at[slot])

    issue(0, 0)

    @pl.loop(0, NCHUNK)
    def _(j):
        par = lax.rem(j, 2)
        gup = gu.at[par]
        gvp = gv.at[par]
        pltpu.make_async_copy(su.at[srcall.at[j]], gup, semu.at[par]).wait()
        pltpu.make_async_copy(sv.at[dstall.at[j]], gvp, semv.at[par]).wait()

        @pl.when(j + 1 < NCHUNK)
        def _():
            issue(j + 1, 1 - par)

        @plsc.parallel_loop(0, NGRP, step=1, unroll=2)
        def _(g):
            evec = iot + g * LL
            accs = [jnp.zeros((LL,), jnp.float32) for _ in range(4)]
            for k in range(NOUT):
                kvec = jnp.full((LL,), k, jnp.int32)
                zu = plsc.load_gather(gup, [evec, kvec])
                zv = plsc.load_gather(gvp, [evec, kvec])
                z = zu + zv
                l = jnp.maximum(z, z * SLOPE)
                accs[k % 4] = accs[k % 4] + avals[k] * l
            acc = (accs[0] + accs[1]) + (accs[2] + accs[3])
            exw[j, pl.ds(g * LL, LL)] = jnp.exp(acc)
        pltpu.async_copy(exw.at[j], shared.at[srcall.at[j]], semsc, add=True)

        @pl.when(j >= 2)
        def _():
            jm = j - 2
            pltpu.make_async_copy(exw.at[jm], shared.at[srcall.at[jm]],
                                  semsc).wait()

    @pl.loop(NCHUNK - 2, NCHUNK)
    def _(j):
        pltpu.make_async_copy(exw.at[j], shared.at[srcall.at[j]], semsc).wait()

    pltpu.sync_copy(exw, ex_hbm.at[wid])
    plsc.subcore_barrier()

    @pl.when(s == 0)
    def _():
        pltpu.sync_copy(shared, parts_hbm.at[c])


def _edge_pass(u, v, src3, dst3, arep):
    return pl.kernel(
        _edge_body,
        out_type=[
            jax.ShapeDtypeStruct((NW, NCHUNK, CHUNK), jnp.float32),
            jax.ShapeDtypeStruct((NC, N), jnp.float32),
        ],
        mesh=_MESH,
        compiler_params=pltpu.CompilerParams(needs_layout_passes=False,
                                             use_tc_tiling_on_sc=False),
        scratch_types=[
            pltpu.VMEM((NCHUNK, CHUNK), jnp.int32),
            pltpu.VMEM((NCHUNK, CHUNK), jnp.int32),
            pltpu.VMEM((2, CHUNK, NOUT), jnp.float32),
            pltpu.VMEM((2, CHUNK, NOUT), jnp.float32),
            pltpu.VMEM((NCHUNK, CHUNK), jnp.float32),
            pltpu.VMEM((NOUT, LL), jnp.float32),
            pltpu.VMEM((CHUNK,), jnp.float32),
            pltpu.VMEM_SHARED((N,), jnp.float32),
            pltpu.VMEM_SHARED((N, NOUT), jnp.float32),
            pltpu.VMEM_SHARED((N, NOUT), jnp.float32),
            pltpu.SemaphoreType.DMA((2,)),
            pltpu.SemaphoreType.DMA((2,)),
            pltpu.SemaphoreType.DMA,
        ],
    )(u, v, src3, dst3, arep)


def _norm_body(ex_hbm, src_hbm, parts_hbm, attn_hbm,
               tab, tmp, srcall, exall, oall, s0, s1, s2, s3):
    c = lax.axis_index("c")
    s = lax.axis_index("s")
    wid = s * NC + c

    c0 = pltpu.async_copy(parts_hbm.at[0], tab, s0)
    c1 = pltpu.async_copy(parts_hbm.at[1], tmp, s1)
    c2 = pltpu.async_copy(src_hbm.at[wid], srcall, s2)
    c3 = pltpu.async_copy(ex_hbm.at[wid], exall, s3)
    c0.wait()
    c1.wait()

    @pl.loop(0, N // LL)
    def _(i):
        sl = pl.ds(i * LL, LL)
        tab[sl] = tab[sl] + tmp[sl]

    c2.wait()
    c3.wait()

    @pl.loop(0, NCHUNK)
    def _(j):
        for g in range(NGRP):
            sl = pl.ds(g * LL, LL)
            idx = srcall[j, sl]
            sv = plsc.load_gather(tab, [idx])
            oall[j, sl] = exall[j, sl] / sv

    pltpu.sync_copy(oall, attn_hbm.at[wid])


def _norm_pass(ex3, src3, parts):
    return pl.kernel(
        _norm_body,
        out_type=jax.ShapeDtypeStruct((NW, NCHUNK, CHUNK), jnp.float32),
        mesh=_MESH,
        compiler_params=pltpu.CompilerParams(needs_layout_passes=False,
                                             use_tc_tiling_on_sc=False),
        scratch_types=[
            pltpu.VMEM((N,), jnp.float32),
            pltpu.VMEM((N,), jnp.float32),
            pltpu.VMEM((NCHUNK, CHUNK), jnp.int32),
            pltpu.VMEM((NCHUNK, CHUNK), jnp.float32),
            pltpu.VMEM((NCHUNK, CHUNK), jnp.float32),
            pltpu.SemaphoreType.DMA,
            pltpu.SemaphoreType.DMA,
            pltpu.SemaphoreType.DMA,
            pltpu.SemaphoreType.DMA,
        ],
    )(ex3, src3, parts)


def kernel(x, edge_index, W_w, W_b, a_w):
    src3 = edge_index[0].reshape(NW, NCHUNK, CHUNK)
    dst3 = edge_index[1].reshape(NW, NCHUNK, CHUNK)
    w_cat = jnp.concatenate([W_w[:, :D].T, W_w[:, D:].T], axis=1)
    b2d = W_b.reshape(1, NOUT)
    arep = jnp.broadcast_to(a_w.reshape(NOUT, 1), (NOUT, LL))
    u, v = _make_uv(x, w_cat, b2d)
    ex3, parts = _edge_pass(u, v, src3, dst3, arep)
    return _norm_pass(ex3, src3, parts).reshape(E)


# speedup vs baseline: 3.6986x; 3.6986x over previous
import jax
import jax.numpy as jnp
from jax import lax
from jax.experimental import pallas as pl
from jax.experimental.pallas import tpu as pltpu
from jax.experimental.pallas import tpu_sc as plsc

N = 10000
E = 320000
D = 128
NOUT = 32
SLOPE = 0.2

NC = 2
NS = 16
LL = 16
NW = NC * NS
EPW = E // NW
CHUNK = 400
NCHUNK = EPW // CHUNK
NGRP = CHUNK // LL


def _uv_body(x_ref, w_ref, b_ref, u_ref, v_ref):
    uv = lax.dot_general(x_ref[...], w_ref[...], (((1,), (0,)), ((), ())),
                         preferred_element_type=jnp.float32)
    u_ref[...] = uv[:, :NOUT] + b_ref[...]
    v_ref[...] = uv[:, NOUT:]


def _make_uv(x, w_cat, b2d):
    blk = 1000
    return pl.pallas_call(
        _uv_body,
        grid=(N // blk,),
        in_specs=[
            pl.BlockSpec((blk, D), lambda i: (i, 0)),
            pl.BlockSpec((D, 2 * NOUT), lambda i: (0, 0)),
            pl.BlockSpec((1, NOUT), lambda i: (0, 0)),
        ],
        out_specs=[
            pl.BlockSpec((blk, NOUT), lambda i: (i, 0)),
            pl.BlockSpec((blk, NOUT), lambda i: (i, 0)),
        ],
        out_shape=[
            jax.ShapeDtypeStruct((N, NOUT), jnp.float32),
            jax.ShapeDtypeStruct((N, NOUT), jnp.float32),
        ],
    )(x, w_cat, b2d)


_MESH = plsc.VectorSubcoreMesh(core_axis_name="c", subcore_axis_name="s",
                               num_cores=NC, num_subcores=NS)


def _edge_body(u_hbm, v_hbm, src_hbm, dst_hbm, arep_hbm,
               ex_hbm, parts_hbm,
               srcall, dstall, gu, gv, exw, arv, zb, shared, su, sv,
               semu, semv, semsc):
    c = lax.axis_index("c")
    s = lax.axis_index("s")
    wid = s * NC + c

    rpt = N // NS
    cu0 = pltpu.async_copy(u_hbm.at[pl.ds(s * rpt, rpt)],
                           su.at[pl.ds(s * rpt, rpt)], semu.at[0])
    cv0 = pltpu.async_copy(v_hbm.at[pl.ds(s * rpt, rpt)],
                           sv.at[pl.ds(s * rpt, rpt)], semv.at[0])

    @pl.when(s == 0)
    def _():
        @pl.loop(0, CHUNK // LL)
        def _(i):
            zb[pl.ds(i * LL, LL)] = jnp.zeros((LL,), jnp.float32)

        @pl.loop(0, N // CHUNK)
        def _(i):
            pltpu.sync_copy(zb, shared.at[pl.ds(i * CHUNK, CHUNK)])

    pltpu.sync_copy(arep_hbm, arv)
    pltpu.sync_copy(src_hbm.at[wid], srcall)
    pltpu.sync_copy(dst_hbm.at[wid], dstall)
    cu0.wait()
    cv0.wait()
    plsc.subcore_barrier()

    av0 = arv[0, :]
    av1 = arv[1, :]
    iot = lax.iota(jnp.int32, LL)

    def issue(j, slot):
        pltpu.async_copy(su.at[srcall.at[j]], gu.at[slot], semu.at[slot])
        pltpu.async_copy(sv.at[dstall.at[j]], gv.at[slot], semv.at[slot])

    issue(0, 0)

    @pl.loop(0, NCHUNK)
    def _(j):
        par = lax.rem(j, 2)
        gup = gu.at[par]
        gvp = gv.at[par]
        pltpu.make_async_copy(su.at[srcall.at[j]], gup, semu.at[par]).wait()
        pltpu.make_async_copy(sv.at[dstall.at[j]], gvp, semv.at[par]).wait()

        @pl.when(j + 1 < NCHUNK)
        def _():
            issue(j + 1, 1 - par)

        @plsc.parallel_loop(0, NGRP, step=1, unroll=2)
        def _(g):
            base = g * LL
            sc = jnp.zeros((LL,), jnp.float32)
            for i in range(LL):
                e = base + i
                u0 = gup[e, pl.ds(0, LL)]
                u1 = gup[e, pl.ds(LL, LL)]
                v0 = gvp[e, pl.ds(0, LL)]
                v1 = gvp[e, pl.ds(LL, LL)]
                z0 = u0 + v0
                z1 = u1 + v1
                l0 = jnp.maximum(z0, z0 * SLOPE)
                l1 = jnp.maximum(z1, z1 * SLOPE)
                p = av0 * l0 + av1 * l1
                sc = jnp.where(iot == i, jnp.sum(p), sc)
            exw[j, pl.ds(base, LL)] = jnp.exp(sc)
        pltpu.async_copy(exw.at[j], shared.at[srcall.at[j]], semsc, add=True)

        @pl.when(j >= 2)
        def _():
            jm = j - 2
            pltpu.make_async_copy(exw.at[jm], shared.at[srcall.at[jm]],
                                  semsc).wait()

    @pl.loop(NCHUNK - 2, NCHUNK)
    def _(j):
        pltpu.make_async_copy(exw.at[j], shared.at[srcall.at[j]], semsc).wait()

    pltpu.sync_copy(exw, ex_hbm.at[wid])
    plsc.subcore_barrier()

    @pl.when(s == 0)
    def _():
        pltpu.sync_copy(shared, parts_hbm.at[c])


def _edge_pass(u, v, src3, dst3, arep):
    return pl.kernel(
        _edge_body,
        out_type=[
            jax.ShapeDtypeStruct((NW, NCHUNK, CHUNK), jnp.float32),
            jax.ShapeDtypeStruct((NC, N), jnp.float32),
        ],
        mesh=_MESH,
        compiler_params=pltpu.CompilerParams(needs_layout_passes=False,
                                             use_tc_tiling_on_sc=False),
        scratch_types=[
            pltpu.VMEM((NCHUNK, CHUNK), jnp.int32),
            pltpu.VMEM((NCHUNK, CHUNK), jnp.int32),
            pltpu.VMEM((2, CHUNK, NOUT), jnp.float32),
            pltpu.VMEM((2, CHUNK, NOUT), jnp.float32),
            pltpu.VMEM((NCHUNK, CHUNK), jnp.float32),
            pltpu.VMEM((2, LL), jnp.float32),
            pltpu.VMEM((CHUNK,), jnp.float32),
            pltpu.VMEM_SHARED((N,), jnp.float32),
            pltpu.VMEM_SHARED((N, NOUT), jnp.float32),
            pltpu.VMEM_SHARED((N, NOUT), jnp.float32),
            pltpu.SemaphoreType.DMA((2,)),
            pltpu.SemaphoreType.DMA((2,)),
            pltpu.SemaphoreType.DMA,
        ],
    )(u, v, src3, dst3, arep)


def _norm_body(ex_hbm, src_hbm, parts_hbm, attn_hbm,
               tab, tmp, srcall, exall, oall, s0, s1, s2, s3):
    c = lax.axis_index("c")
    s = lax.axis_index("s")
    wid = s * NC + c

    c0 = pltpu.async_copy(parts_hbm.at[0], tab, s0)
    c1 = pltpu.async_copy(parts_hbm.at[1], tmp, s1)
    c2 = pltpu.async_copy(src_hbm.at[wid], srcall, s2)
    c3 = pltpu.async_copy(ex_hbm.at[wid], exall, s3)
    c0.wait()
    c1.wait()

    @pl.loop(0, N // LL)
    def _(i):
        sl = pl.ds(i * LL, LL)
        tab[sl] = tab[sl] + tmp[sl]

    c2.wait()
    c3.wait()

    @pl.loop(0, NCHUNK)
    def _(j):
        for g in range(NGRP):
            sl = pl.ds(g * LL, LL)
            idx = srcall[j, sl]
            sv = plsc.load_gather(tab, [idx])
            oall[j, sl] = exall[j, sl] / sv

    pltpu.sync_copy(oall, attn_hbm.at[wid])


def _norm_pass(ex3, src3, parts):
    return pl.kernel(
        _norm_body,
        out_type=jax.ShapeDtypeStruct((NW, NCHUNK, CHUNK), jnp.float32),
        mesh=_MESH,
        compiler_params=pltpu.CompilerParams(needs_layout_passes=False,
                                             use_tc_tiling_on_sc=False),
        scratch_types=[
            pltpu.VMEM((N,), jnp.float32),
            pltpu.VMEM((N,), jnp.float32),
            pltpu.VMEM((NCHUNK, CHUNK), jnp.int32),
            pltpu.VMEM((NCHUNK, CHUNK), jnp.float32),
            pltpu.VMEM((NCHUNK, CHUNK), jnp.float32),
            pltpu.SemaphoreType.DMA,
            pltpu.SemaphoreType.DMA,
            pltpu.SemaphoreType.DMA,
            pltpu.SemaphoreType.DMA,
        ],
    )(ex3, src3, parts)


def kernel(x, edge_index, W_w, W_b, a_w):
    src3 = edge_index[0].reshape(NW, NCHUNK, CHUNK)
    dst3 = edge_index[1].reshape(NW, NCHUNK, CHUNK)
    w_cat = jnp.concatenate([W_w[:, :D].T, W_w[:, D:].T], axis=1)
    b2d = W_b.reshape(1, NOUT)
    a2 = a_w.reshape(2, LL)
    u, v = _make_uv(x, w_cat, b2d)
    ex3, parts = _edge_pass(u, v, src3, dst3, a2)
    return _norm_pass(ex3, src3, parts).reshape(E)


# speedup vs baseline: 3.7526x; 1.0146x over previous
import jax
import jax.numpy as jnp
from jax import lax
from jax.experimental import pallas as pl
from jax.experimental.pallas import tpu as pltpu
from jax.experimental.pallas import tpu_sc as plsc

N = 10000
E = 320000
D = 128
NOUT = 32
SLOPE = 0.2

NC = 2
NS = 16
LL = 16
NW = NC * NS
EPW = E // NW
CHUNK = 400
NCHUNK = EPW // CHUNK
NGRP = CHUNK // LL


def _uv_body(x_ref, w_ref, b_ref, u_ref, v_ref):
    uv = lax.dot_general(x_ref[...], w_ref[...], (((1,), (0,)), ((), ())),
                         preferred_element_type=jnp.float32)
    u_ref[...] = uv[:, :NOUT] + b_ref[...]
    v_ref[...] = uv[:, NOUT:]


def _make_uv(x, w_cat, b2d):
    blk = 1000
    return pl.pallas_call(
        _uv_body,
        grid=(N // blk,),
        in_specs=[
            pl.BlockSpec((blk, D), lambda i: (i, 0)),
            pl.BlockSpec((D, 2 * NOUT), lambda i: (0, 0)),
            pl.BlockSpec((1, NOUT), lambda i: (0, 0)),
        ],
        out_specs=[
            pl.BlockSpec((blk, NOUT), lambda i: (i, 0)),
            pl.BlockSpec((blk, NOUT), lambda i: (i, 0)),
        ],
        out_shape=[
            jax.ShapeDtypeStruct((N, NOUT), jnp.float32),
            jax.ShapeDtypeStruct((N, NOUT), jnp.float32),
        ],
    )(x, w_cat, b2d)


_MESH = plsc.VectorSubcoreMesh(core_axis_name="c", subcore_axis_name="s",
                               num_cores=NC, num_subcores=NS)


def _edge_body(u_hbm, v_hbm, src_hbm, dst_hbm, arep_hbm,
               ex_hbm, parts_hbm,
               srcall, dstall, gu, gv, exw, arv, zb, shared, su, sv,
               semu, semv, semsc):
    c = lax.axis_index("c")
    s = lax.axis_index("s")
    wid = s * NC + c

    rpt = N // NS
    cu0 = pltpu.async_copy(u_hbm.at[pl.ds(s * rpt, rpt)],
                           su.at[pl.ds(s * rpt, rpt)], semu.at[0])
    cv0 = pltpu.async_copy(v_hbm.at[pl.ds(s * rpt, rpt)],
                           sv.at[pl.ds(s * rpt, rpt)], semv.at[0])

    @pl.when(s == 0)
    def _():
        @pl.loop(0, CHUNK // LL)
        def _(i):
            zb[pl.ds(i * LL, LL)] = jnp.zeros((LL,), jnp.float32)

        @pl.loop(0, N // CHUNK)
        def _(i):
            pltpu.sync_copy(zb, shared.at[pl.ds(i * CHUNK, CHUNK)])

    pltpu.sync_copy(arep_hbm, arv)
    pltpu.sync_copy(src_hbm.at[wid], srcall)
    pltpu.sync_copy(dst_hbm.at[wid], dstall)
    cu0.wait()
    cv0.wait()
    plsc.subcore_barrier()

    av0 = arv[0, :]
    av1 = arv[1, :]
    iot = lax.iota(jnp.int32, LL)

    def issue(j, slot):
        pltpu.async_copy(su.at[srcall.at[j]], gu.at[slot], semu.at[slot])
        pltpu.async_copy(sv.at[dstall.at[j]], gv.at[slot], semv.at[slot])

    issue(0, 0)

    @pl.loop(0, NCHUNK)
    def _(j):
        par = lax.rem(j, 2)
        gup = gu.at[par]
        gvp = gv.at[par]
        pltpu.make_async_copy(su.at[srcall.at[j]], gup, semu.at[par]).wait()
        pltpu.make_async_copy(sv.at[dstall.at[j]], gvp, semv.at[par]).wait()

        @pl.when(j + 1 < NCHUNK)
        def _():
            issue(j + 1, 1 - par)

        @plsc.parallel_loop(0, NGRP, step=1, unroll=2)
        def _(g):
            base = g * LL
            sc = jnp.zeros((LL,), jnp.float32)
            for i in range(LL):
                e = base + i
                u0 = gup[e, pl.ds(0, LL)]
                u1 = gup[e, pl.ds(LL, LL)]
                v0 = gvp[e, pl.ds(0, LL)]
                v1 = gvp[e, pl.ds(LL, LL)]
                z0 = u0 + v0
                z1 = u1 + v1
                l0 = jnp.maximum(z0, z0 * SLOPE)
                l1 = jnp.maximum(z1, z1 * SLOPE)
                p = av0 * l0 + av1 * l1
                sc = jnp.where(iot == i, jnp.sum(p), sc)
            exw[j, pl.ds(base, LL)] = jnp.exp(sc)
        pltpu.async_copy(exw.at[j], shared.at[srcall.at[j]], semsc, add=True)

        @pl.when(j >= 2)
        def _():
            jm = j - 2
            pltpu.make_async_copy(exw.at[jm], shared.at[srcall.at[jm]],
                                  semsc).wait()

    @pl.loop(NCHUNK - 2, NCHUNK)
    def _(j):
        pltpu.make_async_copy(exw.at[j], shared.at[srcall.at[j]], semsc).wait()

    pltpu.sync_copy(exw, ex_hbm.at[wid])
    plsc.subcore_barrier()

    @pl.when(s == 0)
    def _():
        pltpu.sync_copy(shared, parts_hbm.at[c])


def _edge_pass(u, v, src3, dst3, arep):
    return pl.kernel(
        _edge_body,
        out_type=[
            jax.ShapeDtypeStruct((NW, NCHUNK, CHUNK), jnp.float32),
            jax.ShapeDtypeStruct((NC, N), jnp.float32),
        ],
        mesh=_MESH,
        compiler_params=pltpu.CompilerParams(needs_layout_passes=False,
                                             use_tc_tiling_on_sc=False),
        scratch_types=[
            pltpu.VMEM((NCHUNK, CHUNK), jnp.int32),
            pltpu.VMEM((NCHUNK, CHUNK), jnp.int32),
            pltpu.VMEM((2, CHUNK, NOUT), jnp.float32),
            pltpu.VMEM((2, CHUNK, NOUT), jnp.float32),
            pltpu.VMEM((NCHUNK, CHUNK), jnp.float32),
            pltpu.VMEM((2, LL), jnp.float32),
            pltpu.VMEM((CHUNK,), jnp.float32),
            pltpu.VMEM_SHARED((N,), jnp.float32),
            pltpu.VMEM_SHARED((N, NOUT), jnp.float32),
            pltpu.VMEM_SHARED((N, NOUT), jnp.float32),
            pltpu.SemaphoreType.DMA((2,)),
            pltpu.SemaphoreType.DMA((2,)),
            pltpu.SemaphoreType.DMA,
        ],
    )(u, v, src3, dst3, arep)


def _norm_body(ex_hbm, src_hbm, parts_hbm, attn_hbm,
               tab, tmp, srcall, exall, oall, sp0, sp1, s0, s1, s2, s3):
    c = lax.axis_index("c")
    s = lax.axis_index("s")
    wid = s * NC + c

    c2 = pltpu.async_copy(src_hbm.at[wid], srcall, s2)
    c3 = pltpu.async_copy(ex_hbm.at[wid], exall, s3)

    @pl.when(s < NS - 1)
    def _():
        sl = pl.ds(s * 640, 640)
        pltpu.sync_copy(parts_hbm.at[0, sl], sp0.at[sl])
        pltpu.sync_copy(parts_hbm.at[1, sl], sp1.at[sl])

    @pl.when(s == NS - 1)
    def _():
        sl = pl.ds((NS - 1) * 640, N - (NS - 1) * 640)
        pltpu.sync_copy(parts_hbm.at[0, sl], sp0.at[sl])
        pltpu.sync_copy(parts_hbm.at[1, sl], sp1.at[sl])

    plsc.subcore_barrier()
    c0 = pltpu.async_copy(sp0, tab, s0)
    c1 = pltpu.async_copy(sp1, tmp, s1)
    c0.wait()
    c1.wait()

    @pl.loop(0, N // LL)
    def _(i):
        sl = pl.ds(i * LL, LL)
        tab[sl] = tab[sl] + tmp[sl]

    c2.wait()
    c3.wait()

    @pl.loop(0, NCHUNK)
    def _(j):
        for g in range(NGRP):
            sl = pl.ds(g * LL, LL)
            idx = srcall[j, sl]
            sv = plsc.load_gather(tab, [idx])
            oall[j, sl] = exall[j, sl] / sv

    pltpu.sync_copy(oall, attn_hbm.at[wid])


def _norm_pass(ex3, src3, parts):
    return pl.kernel(
        _norm_body,
        out_type=jax.ShapeDtypeStruct((NW, NCHUNK, CHUNK), jnp.float32),
        mesh=_MESH,
        compiler_params=pltpu.CompilerParams(needs_layout_passes=False,
                                             use_tc_tiling_on_sc=False),
        scratch_types=[
            pltpu.VMEM((N,), jnp.float32),
            pltpu.VMEM((N,), jnp.float32),
            pltpu.VMEM((NCHUNK, CHUNK), jnp.int32),
            pltpu.VMEM((NCHUNK, CHUNK), jnp.float32),
            pltpu.VMEM((NCHUNK, CHUNK), jnp.float32),
            pltpu.VMEM_SHARED((N,), jnp.float32),
            pltpu.VMEM_SHARED((N,), jnp.float32),
            pltpu.SemaphoreType.DMA,
            pltpu.SemaphoreType.DMA,
            pltpu.SemaphoreType.DMA,
            pltpu.SemaphoreType.DMA,
        ],
    )(ex3, src3, parts)


def kernel(x, edge_index, W_w, W_b, a_w):
    src3 = edge_index[0].reshape(NW, NCHUNK, CHUNK)
    dst3 = edge_index[1].reshape(NW, NCHUNK, CHUNK)
    w_cat = jnp.concatenate([W_w[:, :D].T, W_w[:, D:].T], axis=1)
    b2d = W_b.reshape(1, NOUT)
    a2 = a_w.reshape(2, LL)
    u, v = _make_uv(x, w_cat, b2d)
    ex3, parts = _edge_pass(u, v, src3, dst3, a2)
    return _norm_pass(ex3, src3, parts).reshape(E)
